# baseline (speedup 1.0000x reference)
import functools

import jax
import jax.numpy as jnp
from jax import lax
from jax.experimental import pallas as pl
from jax.experimental.pallas import tpu as pltpu
from jax.experimental.pallas import tpu_sc as plsc

_T = 8192
_D = 1024


@functools.cache
def _copy_kernel():
    info = plsc.get_sparse_core_info()
    nc, ns = info.num_cores, info.num_subcores
    nw = nc * ns
    rows_per_w = _T // nw

    mesh = plsc.VectorSubcoreMesh(core_axis_name="c", subcore_axis_name="s")

    @functools.partial(
        pl.kernel,
        mesh=mesh,
        out_type=jax.ShapeDtypeStruct((_T, _D), jnp.float32),
    )
    def k(table_hbm, out_hbm):
        wid = lax.axis_index("s") * nc + lax.axis_index("c")
        base = wid * rows_per_w
        pltpu.sync_copy(
            table_hbm.at[pl.ds(base, rows_per_w)],
            out_hbm.at[pl.ds(base, rows_per_w)],
        )

    return k


def kernel(table, ids):
    del ids
    return _copy_kernel()(table)


# speedup vs baseline: 24.8069x; 24.8069x over previous
import functools

import jax
import jax.numpy as jnp
from jax import lax
from jax.experimental import pallas as pl
from jax.experimental.pallas import tpu as pltpu
from jax.experimental.pallas import tpu_sc as plsc

_T = 8192
_D = 1024
_CHUNK_ROWS = 32
_NBUF = 3


@functools.cache
def _copy_kernel():
    info = plsc.get_sparse_core_info()
    nc, ns = info.num_cores, info.num_subcores
    nw = nc * ns
    rows_per_w = _T // nw
    chunks = rows_per_w // _CHUNK_ROWS

    mesh = plsc.VectorSubcoreMesh(core_axis_name="c", subcore_axis_name="s")

    @functools.partial(
        pl.kernel,
        mesh=mesh,
        out_type=jax.ShapeDtypeStruct((_T, _D), jnp.float32),
        scratch_types=(
            [pltpu.VMEM((_NBUF, _CHUNK_ROWS, _D), jnp.float32)]
            + [pltpu.SemaphoreType.DMA] * (2 * _NBUF)
        ),
    )
    def k(table_hbm, out_hbm, buf, *sems):
        in_sems, out_sems = sems[:_NBUF], sems[_NBUF:]
        wid = lax.axis_index("s") * nc + lax.axis_index("c")
        base = wid * rows_per_w

        def rng(c):
            return pl.ds(base + c * _CHUNK_ROWS, _CHUNK_ROWS)

        hin = [None] * chunks
        hout = [None] * chunks
        for c in range(_NBUF):
            hin[c] = pltpu.async_copy(table_hbm.at[rng(c)], buf.at[c], in_sems[c])
        for c in range(chunks):
            b = c % _NBUF
            hin[c].wait()
            hout[c] = pltpu.async_copy(buf.at[b], out_hbm.at[rng(c)], out_sems[b])
            nxt = c + _NBUF
            if nxt < chunks:
                hout[c].wait()
                hin[nxt] = pltpu.async_copy(
                    table_hbm.at[rng(nxt)], buf.at[b], in_sems[b]
                )
        for c in range(max(chunks - _NBUF, 0), chunks):
            hout[c].wait()

    return k


def kernel(table, ids):
    del ids
    return _copy_kernel()(table)
